# baseline (speedup 1.0000x reference)
import functools

import jax
import jax.numpy as jnp
from jax import lax
from jax.experimental import pallas as pl
from jax.experimental.pallas import tpu as pltpu
from jax.experimental.pallas import tpu_sc as plsc

_NC = 2
_NS = 16
_LANES = 16


def _sc_aggregate(x, src, dst, edge_attr):
    n, d = x.shape
    e = src.shape[0]
    nw = _NC * _NS
    assert e % nw == 0
    ew = e // nw
    c_main = 128
    n_chunk = ew // c_main
    c_tail = ew - n_chunk * c_main
    zrows = 64
    rows_sub = -(-n // (_NS * zrows)) * zrows
    n_pad = rows_sub * _NS
    assert d % _LANES == 0
    vecs_per_row = d // _LANES

    mesh = plsc.VectorSubcoreMesh(core_axis_name="c", subcore_axis_name="s")

    def body(x_hbm, src_hbm, dst_hbm, ea_hbm, part_hbm,
             srcv, dstv, xv, ev, srcv_t, dstv_t, xv_t, ev_t, zb, aggr, sem):
        cid = lax.axis_index("c")
        sid = lax.axis_index("s")
        wid = sid * _NC + cid

        zero16 = jnp.zeros((_LANES,), jnp.float32)

        def zrow(i, carry):
            for j in range(vecs_per_row):
                zb[i, pl.ds(j * _LANES, _LANES)] = zero16
            return carry

        lax.fori_loop(0, zrows, zrow, 0)
        row0 = sid * rows_sub
        for r in range(rows_sub // zrows):
            pltpu.sync_copy(zb, aggr.at[pl.ds(row0 + r * zrows, zrows), :])
        plsc.subcore_barrier()

        base0 = wid * ew

        def relu_rows(nrows, xbuf, ebuf):
            def rowfn(i, carry):
                for j in range(vecs_per_row):
                    sl = pl.ds(j * _LANES, _LANES)
                    ebuf[i, sl] = jnp.maximum(xbuf[i, sl] + ebuf[i, sl], 0.0)
                return carry
            lax.fori_loop(0, nrows, rowfn, 0)

        def chunk(t, carry):
            base = base0 + t * c_main
            pltpu.sync_copy(src_hbm.at[pl.ds(base, c_main)], srcv)
            pltpu.sync_copy(dst_hbm.at[pl.ds(base, c_main)], dstv)
            pltpu.sync_copy(ea_hbm.at[pl.ds(base, c_main), :], ev)
            pltpu.async_copy(x_hbm.at[srcv], ev, sem, add=True).wait()
            def rowfn(i, carry2):
                for j in range(vecs_per_row):
                    sl = pl.ds(j * _LANES, _LANES)
                    ev[i, sl] = jnp.maximum(ev[i, sl], 0.0)
                return carry2
            lax.fori_loop(0, c_main, rowfn, 0)
            pltpu.sync_copy(ev, aggr.at[dstv], add=True)
            return carry

        lax.fori_loop(0, n_chunk, chunk, 0)

        if c_tail:
            base = base0 + n_chunk * c_main
            pltpu.sync_copy(src_hbm.at[pl.ds(base, c_tail)], srcv_t)
            pltpu.sync_copy(dst_hbm.at[pl.ds(base, c_tail)], dstv_t)
            gat = pltpu.async_copy(x_hbm.at[srcv_t], xv_t, sem)
            pltpu.sync_copy(ea_hbm.at[pl.ds(base, c_tail), :], ev_t)
            gat.wait()
            relu_rows(c_tail, xv_t, ev_t)
            pltpu.sync_copy(ev_t, aggr.at[dstv_t], add=True)

        plsc.subcore_barrier()
        pltpu.sync_copy(aggr.at[pl.ds(row0, rows_sub), :],
                        part_hbm.at[pl.ds(cid * n_pad + row0, rows_sub), :])

    run = pl.kernel(
        body,
        out_type=jax.ShapeDtypeStruct((_NC * n_pad, d), jnp.float32),
        mesh=mesh,
        scratch_types=[
            pltpu.VMEM((c_main,), jnp.int32),
            pltpu.VMEM((c_main,), jnp.int32),
            pltpu.VMEM((c_main, d), jnp.float32),
            pltpu.VMEM((c_main, d), jnp.float32),
            pltpu.VMEM((max(c_tail, 8),), jnp.int32),
            pltpu.VMEM((max(c_tail, 8),), jnp.int32),
            pltpu.VMEM((max(c_tail, 8), d), jnp.float32),
            pltpu.VMEM((max(c_tail, 8), d), jnp.float32),
            pltpu.VMEM((zrows, d), jnp.float32),
            pltpu.VMEM_SHARED((n_pad, d), jnp.float32),
            pltpu.SemaphoreType.DMA,
        ],
    )
    part = run(x, src, dst, edge_attr)
    return part[:n], part[n_pad:n_pad + n]


def _tc_body(x_ref, p0_ref, p1_ref, w1_ref, b1_ref, w2_ref, b2_ref,
             g_ref, bt_ref, out_ref):
    x = x_ref[...]
    z = x + p0_ref[...] + p1_ref[...]
    h = jnp.dot(z, w1_ref[...], preferred_element_type=jnp.float32) + b1_ref[...]
    h = jnp.maximum(h, 0.0)
    h = jnp.dot(h, w2_ref[...], preferred_element_type=jnp.float32) + b2_ref[...]
    r = x + jnp.maximum(h, 0.0)
    mean = jnp.mean(r, axis=1, keepdims=True)
    cen = r - mean
    var = jnp.mean(cen * cen, axis=1, keepdims=True)
    out_ref[...] = cen * lax.rsqrt(var + 1e-5) * g_ref[...] + bt_ref[...]


def _tc_mlp(x, p0, p1, W1, b1, W2, b2, gamma, beta):
    n, d = x.shape
    bn = 2000
    assert n % bn == 0
    grid = n // bn
    row_spec = pl.BlockSpec((bn, d), lambda i: (i, 0))
    full_spec = pl.BlockSpec((d, d), lambda i: (0, 0))
    vec_spec = pl.BlockSpec((1, d), lambda i: (0, 0))
    return pl.pallas_call(
        _tc_body,
        grid=(grid,),
        in_specs=[row_spec, row_spec, row_spec, full_spec, vec_spec,
                  full_spec, vec_spec, vec_spec, vec_spec],
        out_specs=row_spec,
        out_shape=jax.ShapeDtypeStruct((n, d), jnp.float32),
    )(x, p0, p1, W1, b1.reshape(1, d), W2, b2.reshape(1, d),
      gamma.reshape(1, d), beta.reshape(1, d))


def kernel(x, edge_index, edge_attr, W1, b1, W2, b2, gamma, beta):
    ei = edge_index.astype(jnp.int32)
    p0, p1 = _sc_aggregate(x, ei[0], ei[1], edge_attr)
    return _tc_mlp(x, p0, p1, W1, b1, W2, b2, gamma, beta)


# speedup vs baseline: 1.6143x; 1.6143x over previous
import jax
import jax.numpy as jnp
from jax import lax
from jax.experimental import pallas as pl
from jax.experimental.pallas import tpu as pltpu
from jax.experimental.pallas import tpu_sc as plsc

_NC = 2
_NS = 16
_LANES = 16


def _sc_aggregate(x, src, dst, edge_attr):
    n, d = x.shape
    e = src.shape[0]
    nw = _NC * _NS
    c = 128
    rows_total = e // c
    assert rows_total * c == e
    rpw = rows_total // nw
    extra = rows_total - rpw * nw
    nblk = 3
    bchunks = rpw // nblk
    assert bchunks * nblk == rpw and bchunks % 2 == 0
    rows_sub = -(-n // (_NS * c)) * c
    n_pad = rows_sub * _NS
    assert d % _LANES == 0
    vecs_per_row = d // _LANES

    mesh = plsc.VectorSubcoreMesh(core_axis_name="c", subcore_axis_name="s")

    def body(x_hbm, src_hbm, dst_hbm, ea_hbm, part_hbm,
             ev_a, ev_b, srcblk, dstblk, dch_a, dch_b, srcch, aggr,
             se_a, se_b, sg_a, sg_b, ss_a, ss_b):
        cid = lax.axis_index("c")
        sid = lax.axis_index("s")
        wid = sid * _NC + cid

        zero16 = jnp.zeros((_LANES,), jnp.float32)

        def relu_rows(ebuf):
            def rowfn(i, carry):
                for j in range(vecs_per_row):
                    sl = pl.ds(j * _LANES, _LANES)
                    ebuf[i, sl] = jnp.maximum(ebuf[i, sl], 0.0)
                return carry
            lax.fori_loop(0, c, rowfn, 0)

        def copy_idx_row(blk, t, dch):
            def vfn(j, carry):
                dch[pl.ds(j * _LANES, _LANES)] = blk[pl.ds(t * c + j * _LANES,
                                                           _LANES)]
                return carry
            lax.fori_loop(0, c // _LANES, vfn, 0)

        def zrow(i, carry):
            for j in range(vecs_per_row):
                ev_a[i, pl.ds(j * _LANES, _LANES)] = zero16
            return carry

        lax.fori_loop(0, c, zrow, 0)
        row0 = sid * rows_sub
        for r in range(rows_sub // c):
            pltpu.sync_copy(ev_a, aggr.at[pl.ds(row0 + r * c, c), :])
        plsc.subcore_barrier()

        base_row = wid * rpw

        def ea_issue(row, ev, sem):
            return pltpu.async_copy(ea_hbm.at[pl.ds(row * c, c), :], ev, sem)

        def process(t, ev, sem_g):
            gat = pltpu.async_copy(
                x_hbm.at[srcblk.at[pl.ds(t * c, c)]], ev, sem_g, add=True)
            return gat

        for b in range(nblk):
            brow = base_row + b * bchunks
            ebase = brow * c
            pltpu.sync_copy(src_hbm.at[pl.ds(ebase, bchunks * c)], srcblk)
            pltpu.sync_copy(dst_hbm.at[pl.ds(ebase, bchunks * c)], dstblk)
            ea_issue(brow, ev_a, se_a)
            ea_issue(brow + 1, ev_b, se_b)

            def pair(p, carry):
                t0 = 2 * p
                t1 = t0 + 1
                pltpu.make_async_copy(ea_hbm.at[pl.ds(0, c), :], ev_a,
                                      se_a).wait()
                ga = process(t0, ev_a, sg_a)
                pltpu.make_async_copy(ea_hbm.at[pl.ds(0, c), :], ev_b,
                                      se_b).wait()
                gb = process(t1, ev_b, sg_b)
                ga.wait()
                relu_rows(ev_a)
                copy_idx_row(dstblk, t0, dch_a)
                pltpu.async_copy(ev_a, aggr.at[dch_a], ss_a, add=True)
                gb.wait()
                relu_rows(ev_b)
                copy_idx_row(dstblk, t1, dch_b)
                pltpu.async_copy(ev_b, aggr.at[dch_b], ss_b, add=True)

                @pl.when(p < bchunks // 2 - 1)
                def _prefetch():
                    pltpu.make_async_copy(ev_a, aggr.at[dch_a], ss_a).wait()
                    ea_issue(brow + t0 + 2, ev_a, se_a)
                    pltpu.make_async_copy(ev_b, aggr.at[dch_b], ss_b).wait()
                    ea_issue(brow + t1 + 2, ev_b, se_b)

                return carry

            lax.fori_loop(0, bchunks // 2, pair, 0)
            pltpu.make_async_copy(ev_a, aggr.at[dch_a], ss_a).wait()
            pltpu.make_async_copy(ev_b, aggr.at[dch_b], ss_b).wait()

        @pl.when(wid < extra)
        def _leftover():
            ebase = (nw * rpw + wid) * c
            pltpu.sync_copy(src_hbm.at[pl.ds(ebase, c)], srcch)
            pltpu.sync_copy(dst_hbm.at[pl.ds(ebase, c)], dch_a)
            pltpu.sync_copy(ea_hbm.at[pl.ds(ebase, c), :], ev_a)
            pltpu.async_copy(x_hbm.at[srcch], ev_a, sg_a, add=True).wait()
            relu_rows(ev_a)
            pltpu.sync_copy(ev_a, aggr.at[dch_a], add=True)

        plsc.subcore_barrier()
        pltpu.sync_copy(aggr.at[pl.ds(row0, rows_sub), :],
                        part_hbm.at[pl.ds(cid * n_pad + row0, rows_sub), :])

    run = pl.kernel(
        body,
        out_type=jax.ShapeDtypeStruct((_NC * n_pad, d), jnp.float32),
        mesh=mesh,
        scratch_types=[
            pltpu.VMEM((c, d), jnp.float32),
            pltpu.VMEM((c, d), jnp.float32),
            pltpu.VMEM((bchunks * c,), jnp.int32),
            pltpu.VMEM((bchunks * c,), jnp.int32),
            pltpu.VMEM((c,), jnp.int32),
            pltpu.VMEM((c,), jnp.int32),
            pltpu.VMEM((c,), jnp.int32),
            pltpu.VMEM_SHARED((n_pad, d), jnp.float32),
            pltpu.SemaphoreType.DMA,
            pltpu.SemaphoreType.DMA,
            pltpu.SemaphoreType.DMA,
            pltpu.SemaphoreType.DMA,
            pltpu.SemaphoreType.DMA,
            pltpu.SemaphoreType.DMA,
        ],
    )
    part = run(x, src, dst, edge_attr)
    return part[:n], part[n_pad:n_pad + n]


def _tc_body(x_ref, p0_ref, p1_ref, w1_ref, b1_ref, w2_ref, b2_ref,
             g_ref, bt_ref, out_ref):
    x = x_ref[...]
    z = x + p0_ref[...] + p1_ref[...]
    h = jnp.dot(z, w1_ref[...], preferred_element_type=jnp.float32) + b1_ref[...]
    h = jnp.maximum(h, 0.0)
    h = jnp.dot(h, w2_ref[...], preferred_element_type=jnp.float32) + b2_ref[...]
    r = x + jnp.maximum(h, 0.0)
    mean = jnp.mean(r, axis=1, keepdims=True)
    cen = r - mean
    var = jnp.mean(cen * cen, axis=1, keepdims=True)
    out_ref[...] = cen * lax.rsqrt(var + 1e-5) * g_ref[...] + bt_ref[...]


def _tc_mlp(x, p0, p1, W1, b1, W2, b2, gamma, beta):
    n, d = x.shape
    bn = 2000
    assert n % bn == 0
    grid = n // bn
    row_spec = pl.BlockSpec((bn, d), lambda i: (i, 0))
    full_spec = pl.BlockSpec((d, d), lambda i: (0, 0))
    vec_spec = pl.BlockSpec((1, d), lambda i: (0, 0))
    return pl.pallas_call(
        _tc_body,
        grid=(grid,),
        in_specs=[row_spec, row_spec, row_spec, full_spec, vec_spec,
                  full_spec, vec_spec, vec_spec, vec_spec],
        out_specs=row_spec,
        out_shape=jax.ShapeDtypeStruct((n, d), jnp.float32),
    )(x, p0, p1, W1, b1.reshape(1, d), W2, b2.reshape(1, d),
      gamma.reshape(1, d), beta.reshape(1, d))


def kernel(x, edge_index, edge_attr, W1, b1, W2, b2, gamma, beta):
    ei = edge_index.astype(jnp.int32)
    p0, p1 = _sc_aggregate(x, ei[0], ei[1], edge_attr)
    return _tc_mlp(x, p0, p1, W1, b1, W2, b2, gamma, beta)
